# baseline (speedup 1.0000x reference)
import jax
import jax.numpy as jnp
from jax import lax
from jax.experimental import pallas as pl
from jax.experimental.pallas import tpu as pltpu
from jax.experimental.pallas import tpu_sc as plsc

_B = 16384
_D = 32
_L = 16
_NW = 32
_NU = 1000000
_NB = 100000
_PC = 1024
_SCAN = 2048
_SUB = 2048
_ACC = 16640


def _wid():
    return lax.axis_index("s") * 2 + lax.axis_index("c")


def _range_bounds(wid, total):
    per = total // _NW
    lo = jnp.bitwise_and(per * wid, ~127)
    hi = jnp.where(wid == _NW - 1, total, jnp.bitwise_and(per * (wid + 1), ~127))
    return lo, hi


def _scan_compact(idx_hbm, lo, hi, stag_v, list_v, iota):
    width = hi - lo

    def chunk_body(cb, cnt):
        pltpu.sync_copy(idx_hbm.at[pl.ds(cb * _SCAN, _SCAN)], stag_v)

        def vec_body(k, cnt):
            v = stag_v[pl.ds(k * _L, _L)]
            rel = v - lo
            mask = plsc.bitcast(rel, jnp.uint32) < plsc.bitcast(
                jnp.full((_L,), width, jnp.int32), jnp.uint32)
            pos = cb * _SCAN + k * _L + iota
            packed = jnp.bitwise_or(lax.shift_left(rel, 15), pos)
            plsc.store_compressed(list_v.at[pl.ds(cnt, _L)], packed, mask=mask)
            pc = plsc.all_reduce_population_count(mask)
            return cnt + pc[0]

        return lax.fori_loop(0, _SCAN // _L, vec_body, cnt)

    return lax.fori_loop(0, _B // _SCAN, chunk_body, 0)


def _window(list_v, sub_v, src_v, acc_sh, wvec, iota, n,
            rel_lo, rel_w, pan, col_off, col_hi):
    nvec = lax.div(n + (_L - 1), _L)
    dump = jnp.full((_L,), _B, jnp.int32)

    def blk_body(blk, carry):
        base = blk * _SUB

        def sub_body(k, cnt2):
            pk = list_v[pl.ds(base + k * _L, _L)]
            r2 = lax.shift_right_logical(pk, 15) - rel_lo
            mask = plsc.bitcast(r2, jnp.uint32) < plsc.bitcast(
                jnp.full((_L,), rel_w, jnp.int32), jnp.uint32)
            valid = (base + k * _L + iota) < n
            mask = jnp.logical_and(mask, valid)
            plsc.store_compressed(sub_v.at[pl.ds(cnt2, _L)], pk, mask=mask)
            pc = plsc.all_reduce_population_count(mask)
            return cnt2 + pc[0]

        kvecs = jnp.minimum(nvec - blk * (_SUB // _L), _SUB // _L)
        cnt2 = lax.fori_loop(0, kvecs, sub_body, 0)
        sub_v[pl.ds(cnt2, _L)] = dump

        def grp_body(g, carry):
            pk = sub_v[pl.ds(g * _L, _L)]
            pos = jnp.bitwise_and(pk, 32767)
            col = lax.shift_right_logical(pk, 15) - rel_lo + col_off
            col = jnp.minimum(jnp.maximum(col, 0), col_hi)
            acc = jnp.zeros((_L,), jnp.float32)
            for j in range(_D):
                pv = plsc.load_gather(pan, [jnp.full((_L,), j, jnp.int32), col])
                acc = acc + pv * wvec[j]
            src_v[...] = acc
            pltpu.sync_copy(src_v, acc_sh.at[pos], add=True)
            return carry

        lax.fori_loop(0, lax.div(cnt2 + (_L - 1), _L), grp_body, 0)
        return carry

    lax.fori_loop(0, lax.div(n + (_SUB - 1), _SUB), blk_body, 0)


def _panel_pass(tab_hbm, lo, hi, tail_w, n, list_v, panel_v, sub_v, src_v,
                acc_sh, wvec, psem, iota):
    hi_main = jnp.bitwise_and(hi, ~127)
    n_panels = lax.div(hi_main - lo + (_PC - 1), _PC)

    def fire(p, slot):
        start = pl.multiple_of(jnp.minimum(lo + p * _PC, hi_main - _PC), 128)
        for jt in range(_D // 8):
            for ct in range(_PC // 128):
                pltpu.async_copy(
                    tab_hbm.at[pl.ds(jt * 8, 8),
                               pl.ds(start + ct * 128, 128)],
                    panel_v.at[slot, pl.ds(jt * 8, 8), pl.ds(ct * 128, 128)],
                    psem)

    def drain(slot):
        for jt in range(_D // 8):
            for ct in range(_PC // 128):
                pltpu.make_async_copy(
                    tab_hbm.at[pl.ds(jt * 8, 8), pl.ds(0, 128)],
                    panel_v.at[slot, pl.ds(jt * 8, 8), pl.ds(ct * 128, 128)],
                    psem).wait()

    fire(0, 0)
    drain(0)

    def panel_body(p, carry):
        @pl.when(p + 1 < n_panels)
        def _():
            fire(p + 1, (p + 1) % 2)

        start = jnp.minimum(lo + p * _PC, hi_main - _PC)
        rel_lo = p * _PC
        rel_w = jnp.minimum(rel_lo + _PC, hi_main - lo) - rel_lo
        with jax.named_scope("window"):
            _window(list_v, sub_v, src_v, acc_sh, wvec, iota, n,
                    rel_lo, rel_w, panel_v.at[p % 2],
                    (lo + rel_lo) - start, _PC - 1)
        @pl.when(p + 1 < n_panels)
        def _():
            with jax.named_scope("drain"):
                drain((p + 1) % 2)
        return carry

    lax.fori_loop(0, n_panels, panel_body, 0)

    if tail_w:
        @pl.when(hi != hi_main)
        def _():
            tstart = pl.multiple_of(hi_main, 128)
            for jt in range(_D // 8):
                pltpu.sync_copy(
                    tab_hbm.at[pl.ds(jt * 8, 8), pl.ds(tstart, 128)],
                    panel_v.at[0, pl.ds(jt * 8, 8), pl.ds(0, 128)])
            _window(list_v, sub_v, src_v, acc_sh, wvec, iota, n,
                    hi_main - lo, hi - hi_main, panel_v.at[0],
                    0, 127)


def _phase1(users_hbm, books_hbm, ut_hbm, bt_hbm, wb_hbm,
            outp_hbm,
            stag_v, ulist_v, blist_v, panel_v, sub_v, src_v, w_v, zbuf_v,
            acc_sh, psem):
    wid = _wid()
    sid = lax.axis_index("s")
    core = lax.axis_index("c")
    ulo, uhi = _range_bounds(wid, _NU)
    blo, bhi = _range_bounds(wid, _NB)

    pltpu.sync_copy(wb_hbm, w_v)
    iota = lax.iota(jnp.int32, _L)

    def z_body(k, carry):
        zbuf_v[pl.ds(k * _L, _L)] = jnp.zeros((_L,), jnp.float32)
        return carry
    lax.fori_loop(0, (_ACC // 16) // _L, z_body, 0)
    pltpu.sync_copy(zbuf_v, acc_sh.at[pl.ds(pl.multiple_of(sid * (_ACC // 16), 16), _ACC // 16)])
    plsc.subcore_barrier()

    with jax.named_scope("scan_u"):
        nu = _scan_compact(users_hbm, ulo, uhi, stag_v, ulist_v, iota)
    with jax.named_scope("scan_b"):
        nb = _scan_compact(books_hbm, blo, bhi, stag_v, blist_v, iota)

    wvecs = [w_v[pl.ds(k * _L, _L)] for k in range(4)]
    wu = [wvecs[j // _L][j % _L] for j in range(_D)]
    wk = [wvecs[2 + j // _L][j % _L] for j in range(_D)]

    with jax.named_scope("upass"):
        _panel_pass(ut_hbm, ulo, uhi, _NU % 128, nu, ulist_v, panel_v, sub_v,
                    src_v, acc_sh, wu, psem, iota)
    with jax.named_scope("bpass"):
        _panel_pass(bt_hbm, blo, bhi, _NB % 128, nb, blist_v, panel_v, sub_v,
                    src_v, acc_sh, wk, psem, iota)

    plsc.subcore_barrier()
    off = pl.multiple_of(sid * (_ACC // 16), 16)
    pltpu.sync_copy(acc_sh.at[pl.ds(off, _ACC // 16)], zbuf_v)
    pltpu.sync_copy(
        zbuf_v,
        outp_hbm.at[pl.ds(pl.multiple_of(core * _ACC + off, 16), _ACC // 16)])


def _phase2(outp_hbm, wb_hbm, out_hbm, u_v, b_v, w_v):
    wid = _wid()
    base = wid * (_B // _NW)
    pltpu.sync_copy(outp_hbm.at[pl.ds(base, _B // _NW)], u_v)
    pltpu.sync_copy(outp_hbm.at[pl.ds(_ACC + base, _B // _NW)], b_v)
    pltpu.sync_copy(wb_hbm, w_v)
    bias = w_v[pl.ds(2 * _D, _L)][0]

    def body(k, carry):
        sl = pl.ds(k * _L, _L)
        u_v[sl] = u_v[sl] + b_v[sl] + bias
        return carry

    lax.fori_loop(0, (_B // _NW) // _L, body, 0)
    pltpu.sync_copy(u_v, out_hbm.at[pl.ds(base, _B // _NW)])


@jax.jit
def kernel(users, books, user_table, book_table, W, b):
    users = users.astype(jnp.int32)
    books = books.astype(jnp.int32)
    ut = user_table.T
    bt = book_table.T
    wb = jnp.concatenate(
        [W.reshape(-1), jnp.broadcast_to(b.reshape(-1)[0], (16,))]
    ).astype(jnp.float32)

    mesh = plsc.VectorSubcoreMesh(core_axis_name="c", subcore_axis_name="s")
    cap = _B + 2 * _L

    p1 = pl.kernel(
        _phase1,
        out_type=jax.ShapeDtypeStruct((2 * _ACC,), jnp.float32),
        mesh=mesh,
        compiler_params=pltpu.CompilerParams(needs_layout_passes=False),
        scratch_types=[
            pltpu.VMEM((_SCAN,), jnp.int32),
            pltpu.VMEM((cap,), jnp.int32),
            pltpu.VMEM((cap,), jnp.int32),
            pltpu.VMEM((2, _D, _PC), jnp.float32),
            pltpu.VMEM((_SUB + _L,), jnp.int32),
            pltpu.VMEM((_L,), jnp.float32),
            pltpu.VMEM((2 * _D + _L,), jnp.float32),
            pltpu.VMEM((_ACC // 16,), jnp.float32),
            pltpu.VMEM_SHARED((_ACC,), jnp.float32),
            pltpu.SemaphoreType.DMA,
        ],
    )
    outp = p1(users, books, ut, bt, wb)

    p2 = pl.kernel(
        _phase2,
        out_type=jax.ShapeDtypeStruct((_B,), jnp.float32),
        mesh=mesh,
        compiler_params=pltpu.CompilerParams(needs_layout_passes=False),
        scratch_types=[
            pltpu.VMEM((_B // _NW,), jnp.float32),
            pltpu.VMEM((_B // _NW,), jnp.float32),
            pltpu.VMEM((2 * _D + _L,), jnp.float32),
        ],
    )
    out = p2(outp, wb)
    return out.reshape(_B, 1)


# speedup vs baseline: 1.0210x; 1.0210x over previous
import jax
import jax.numpy as jnp
from jax import lax
from jax.experimental import pallas as pl
from jax.experimental.pallas import tpu as pltpu
from jax.experimental.pallas import tpu_sc as plsc

_B = 16384
_D = 32
_L = 16
_NW = 32
_NU = 1000000
_NB = 100000
_PC = 1024
_SCAN = 2048
_SUB = 2048
_ACC = 16640


def _wid():
    return lax.axis_index("s") * 2 + lax.axis_index("c")


def _range_bounds(wid, total):
    per = total // _NW
    lo = jnp.bitwise_and(per * wid, ~127)
    hi = jnp.where(wid == _NW - 1, total, jnp.bitwise_and(per * (wid + 1), ~127))
    return lo, hi


def _scan_compact(users_hbm, books_hbm, ulo, uw, blo, bw,
                  stagu_v, stagb_v, ulist_v, blist_v, iota):
    def chunk_body(cb, cnt):
        pltpu.sync_copy(users_hbm.at[pl.ds(cb * _SCAN, _SCAN)], stagu_v)
        pltpu.sync_copy(books_hbm.at[pl.ds(cb * _SCAN, _SCAN)], stagb_v)

        def vec_body(k, cnt):
            cntu, cntb = cnt
            pos = cb * _SCAN + k * _L + iota
            u = stagu_v[pl.ds(k * _L, _L)]
            urel = u - ulo
            umask = plsc.bitcast(urel, jnp.uint32) < plsc.bitcast(
                jnp.full((_L,), uw, jnp.int32), jnp.uint32)
            upk = jnp.bitwise_or(lax.shift_left(urel, 15), pos)
            plsc.store_compressed(ulist_v.at[pl.ds(cntu, _L)], upk, mask=umask)
            b = stagb_v[pl.ds(k * _L, _L)]
            brel = b - blo
            bmask = plsc.bitcast(brel, jnp.uint32) < plsc.bitcast(
                jnp.full((_L,), bw, jnp.int32), jnp.uint32)
            bpk = jnp.bitwise_or(lax.shift_left(brel, 15), pos)
            plsc.store_compressed(blist_v.at[pl.ds(cntb, _L)], bpk, mask=bmask)
            upc = plsc.all_reduce_population_count(umask)
            bpc = plsc.all_reduce_population_count(bmask)
            return (cntu + upc[0], cntb + bpc[0])

        return lax.fori_loop(0, _SCAN // _L, vec_body, cnt)

    return lax.fori_loop(0, _B // _SCAN, chunk_body, (0, 0))


def _window(list_v, sub_v, src_v, acc_sh, wvec, iota, n,
            rel_lo, rel_w, pan, col_off, col_hi):
    nvec = lax.div(n + (_L - 1), _L)
    dump = jnp.full((_L,), _B, jnp.int32)

    def blk_body(blk, carry):
        base = blk * _SUB

        def sub_body(k, cnt2):
            pk = list_v[pl.ds(base + k * _L, _L)]
            r2 = lax.shift_right_logical(pk, 15) - rel_lo
            mask = plsc.bitcast(r2, jnp.uint32) < plsc.bitcast(
                jnp.full((_L,), rel_w, jnp.int32), jnp.uint32)
            valid = (base + k * _L + iota) < n
            mask = jnp.logical_and(mask, valid)
            plsc.store_compressed(sub_v.at[pl.ds(cnt2, _L)], pk, mask=mask)
            pc = plsc.all_reduce_population_count(mask)
            return cnt2 + pc[0]

        kvecs = jnp.minimum(nvec - blk * (_SUB // _L), _SUB // _L)
        cnt2 = lax.fori_loop(0, kvecs, sub_body, 0)
        sub_v[pl.ds(cnt2, _L)] = dump

        def grp_body(g, carry):
            pk = sub_v[pl.ds(g * _L, _L)]
            pos = jnp.bitwise_and(pk, 32767)
            col = lax.shift_right_logical(pk, 15) - rel_lo + col_off
            col = jnp.minimum(jnp.maximum(col, 0), col_hi)
            acc = jnp.zeros((_L,), jnp.float32)
            for j in range(_D):
                pv = plsc.load_gather(pan, [jnp.full((_L,), j, jnp.int32), col])
                acc = acc + pv * wvec[j]
            src_v[...] = acc
            pltpu.sync_copy(src_v, acc_sh.at[pos], add=True)
            return carry

        lax.fori_loop(0, lax.div(cnt2 + (_L - 1), _L), grp_body, 0)
        return carry

    lax.fori_loop(0, lax.div(n + (_SUB - 1), _SUB), blk_body, 0)


def _fire_tiles(tab_hbm, panel_v, psem, lo, hi_main, p, slot):
    start = pl.multiple_of(jnp.minimum(lo + p * _PC, hi_main - _PC), 128)
    for jt in range(_D // 8):
        for ct in range(_PC // 128):
            pltpu.async_copy(
                tab_hbm.at[pl.ds(jt * 8, 8), pl.ds(start + ct * 128, 128)],
                panel_v.at[slot, pl.ds(jt * 8, 8), pl.ds(ct * 128, 128)],
                psem)


def _panel_pass(tab_hbm, lo, hi, tail_w, n, list_v, panel_v, sub_v, src_v,
                acc_sh, wvec, psem, iota, prefired=False):
    hi_main = jnp.bitwise_and(hi, ~127)
    n_panels = lax.div(hi_main - lo + (_PC - 1), _PC)

    def fire(p, slot):
        _fire_tiles(tab_hbm, panel_v, psem, lo, hi_main, p, slot)

    def drain(slot):
        for jt in range(_D // 8):
            for ct in range(_PC // 128):
                pltpu.make_async_copy(
                    tab_hbm.at[pl.ds(jt * 8, 8), pl.ds(0, 128)],
                    panel_v.at[slot, pl.ds(jt * 8, 8), pl.ds(ct * 128, 128)],
                    psem).wait()

    if not prefired:
        fire(0, 0)
    drain(0)

    def panel_body(p, carry):
        fire_pred = p + 1 < n_panels
        if prefired:
            fire_pred = jnp.logical_and(fire_pred, p >= 1)

        @pl.when(fire_pred)
        def _():
            fire(p + 1, (p + 1) % 2)

        start = jnp.minimum(lo + p * _PC, hi_main - _PC)
        rel_lo = p * _PC
        rel_w = jnp.minimum(rel_lo + _PC, hi_main - lo) - rel_lo
        with jax.named_scope("window"):
            _window(list_v, sub_v, src_v, acc_sh, wvec, iota, n,
                    rel_lo, rel_w, panel_v.at[p % 2],
                    (lo + rel_lo) - start, _PC - 1)
        @pl.when(p + 1 < n_panels)
        def _():
            with jax.named_scope("drain"):
                drain((p + 1) % 2)
        return carry

    lax.fori_loop(0, n_panels, panel_body, 0)

    if tail_w:
        @pl.when(hi != hi_main)
        def _():
            tstart = pl.multiple_of(hi_main, 128)
            for jt in range(_D // 8):
                pltpu.sync_copy(
                    tab_hbm.at[pl.ds(jt * 8, 8), pl.ds(tstart, 128)],
                    panel_v.at[0, pl.ds(jt * 8, 8), pl.ds(0, 128)])
            _window(list_v, sub_v, src_v, acc_sh, wvec, iota, n,
                    hi_main - lo, hi - hi_main, panel_v.at[0],
                    0, 127)


def _phase1(users_hbm, books_hbm, ut_hbm, bt_hbm, wb_hbm,
            outp_hbm,
            stag_v, stagb_v, ulist_v, blist_v, panel_v, sub_v, src_v, w_v,
            zbuf_v, acc_sh, psem):
    wid = _wid()
    sid = lax.axis_index("s")
    core = lax.axis_index("c")
    ulo, uhi = _range_bounds(wid, _NU)
    blo, bhi = _range_bounds(wid, _NB)

    pltpu.sync_copy(wb_hbm, w_v)
    iota = lax.iota(jnp.int32, _L)

    uhi_main = jnp.bitwise_and(uhi, ~127)
    _fire_tiles(ut_hbm, panel_v, psem, ulo, uhi_main, 0, 0)
    _fire_tiles(ut_hbm, panel_v, psem, ulo, uhi_main, 1, 1)

    def z_body(k, carry):
        zbuf_v[pl.ds(k * _L, _L)] = jnp.zeros((_L,), jnp.float32)
        return carry
    lax.fori_loop(0, (_ACC // 16) // _L, z_body, 0)
    pltpu.sync_copy(zbuf_v, acc_sh.at[pl.ds(pl.multiple_of(sid * (_ACC // 16), 16), _ACC // 16)])
    plsc.subcore_barrier()

    with jax.named_scope("scan"):
        nu, nb = _scan_compact(users_hbm, books_hbm, ulo, uhi - ulo,
                               blo, bhi - blo, stag_v, stagb_v,
                               ulist_v, blist_v, iota)

    wvecs = [w_v[pl.ds(k * _L, _L)] for k in range(4)]
    wu = [wvecs[j // _L][j % _L] for j in range(_D)]
    wk = [wvecs[2 + j // _L][j % _L] for j in range(_D)]

    with jax.named_scope("upass"):
        _panel_pass(ut_hbm, ulo, uhi, _NU % 128, nu, ulist_v, panel_v, sub_v,
                    src_v, acc_sh, wu, psem, iota, prefired=True)
    with jax.named_scope("bpass"):
        _panel_pass(bt_hbm, blo, bhi, _NB % 128, nb, blist_v, panel_v, sub_v,
                    src_v, acc_sh, wk, psem, iota)

    plsc.subcore_barrier()
    off = pl.multiple_of(sid * (_ACC // 16), 16)
    pltpu.sync_copy(acc_sh.at[pl.ds(off, _ACC // 16)], zbuf_v)
    pltpu.sync_copy(
        zbuf_v,
        outp_hbm.at[pl.ds(pl.multiple_of(core * _ACC + off, 16), _ACC // 16)])


def _phase2(outp_hbm, wb_hbm, out_hbm, u_v, b_v, w_v):
    wid = _wid()
    base = wid * (_B // _NW)
    pltpu.sync_copy(outp_hbm.at[pl.ds(base, _B // _NW)], u_v)
    pltpu.sync_copy(outp_hbm.at[pl.ds(_ACC + base, _B // _NW)], b_v)
    pltpu.sync_copy(wb_hbm, w_v)
    bias = w_v[pl.ds(2 * _D, _L)][0]

    def body(k, carry):
        sl = pl.ds(k * _L, _L)
        u_v[sl] = u_v[sl] + b_v[sl] + bias
        return carry

    lax.fori_loop(0, (_B // _NW) // _L, body, 0)
    pltpu.sync_copy(u_v, out_hbm.at[pl.ds(base, _B // _NW)])


@jax.jit
def kernel(users, books, user_table, book_table, W, b):
    users = users.astype(jnp.int32)
    books = books.astype(jnp.int32)
    ut = user_table.T
    bt = book_table.T
    wb = jnp.concatenate(
        [W.reshape(-1), jnp.broadcast_to(b.reshape(-1)[0], (16,))]
    ).astype(jnp.float32)

    mesh = plsc.VectorSubcoreMesh(core_axis_name="c", subcore_axis_name="s")
    cap = _B + 2 * _L

    p1 = pl.kernel(
        _phase1,
        out_type=jax.ShapeDtypeStruct((2 * _ACC,), jnp.float32),
        mesh=mesh,
        compiler_params=pltpu.CompilerParams(needs_layout_passes=False),
        scratch_types=[
            pltpu.VMEM((_SCAN,), jnp.int32),
            pltpu.VMEM((_SCAN,), jnp.int32),
            pltpu.VMEM((cap,), jnp.int32),
            pltpu.VMEM((cap,), jnp.int32),
            pltpu.VMEM((2, _D, _PC), jnp.float32),
            pltpu.VMEM((_SUB + _L,), jnp.int32),
            pltpu.VMEM((_L,), jnp.float32),
            pltpu.VMEM((2 * _D + _L,), jnp.float32),
            pltpu.VMEM((_ACC // 16,), jnp.float32),
            pltpu.VMEM_SHARED((_ACC,), jnp.float32),
            pltpu.SemaphoreType.DMA,
        ],
    )
    outp = p1(users, books, ut, bt, wb)

    p2 = pl.kernel(
        _phase2,
        out_type=jax.ShapeDtypeStruct((_B,), jnp.float32),
        mesh=mesh,
        compiler_params=pltpu.CompilerParams(needs_layout_passes=False),
        scratch_types=[
            pltpu.VMEM((_B // _NW,), jnp.float32),
            pltpu.VMEM((_B // _NW,), jnp.float32),
            pltpu.VMEM((2 * _D + _L,), jnp.float32),
        ],
    )
    out = p2(outp, wb)
    return out.reshape(_B, 1)


# speedup vs baseline: 1.0249x; 1.0039x over previous
import jax
import jax.numpy as jnp
from jax import lax
from jax.experimental import pallas as pl
from jax.experimental.pallas import tpu as pltpu
from jax.experimental.pallas import tpu_sc as plsc

_B = 16384
_D = 32
_L = 16
_NW = 32
_NU = 1000000
_NB = 100000
_PC = 1024
_SCAN = 2048
_SUB = 2048
_ACC = 16640


def _wid():
    return lax.axis_index("s") * 2 + lax.axis_index("c")


def _range_bounds(wid, total):
    per = total // _NW
    lo = jnp.bitwise_and(per * wid, ~127)
    hi = jnp.where(wid == _NW - 1, total, jnp.bitwise_and(per * (wid + 1), ~127))
    return lo, hi


def _scan_compact(users_hbm, books_hbm, ulo, uw, blo, bw,
                  stagu_v, stagb_v, ulist_v, blist_v, iota):
    def chunk_body(cb, cnt):
        pltpu.sync_copy(users_hbm.at[pl.ds(cb * _SCAN, _SCAN)], stagu_v)
        pltpu.sync_copy(books_hbm.at[pl.ds(cb * _SCAN, _SCAN)], stagb_v)

        def vec_body(k, cnt):
            cntu, cntb = cnt
            pos = cb * _SCAN + k * _L + iota
            u = stagu_v[pl.ds(k * _L, _L)]
            urel = u - ulo
            umask = plsc.bitcast(urel, jnp.uint32) < plsc.bitcast(
                jnp.full((_L,), uw, jnp.int32), jnp.uint32)
            upk = jnp.bitwise_or(lax.shift_left(urel, 15), pos)
            plsc.store_compressed(ulist_v.at[pl.ds(cntu, _L)], upk, mask=umask)
            b = stagb_v[pl.ds(k * _L, _L)]
            brel = b - blo
            bmask = plsc.bitcast(brel, jnp.uint32) < plsc.bitcast(
                jnp.full((_L,), bw, jnp.int32), jnp.uint32)
            bpk = jnp.bitwise_or(lax.shift_left(brel, 15), pos)
            plsc.store_compressed(blist_v.at[pl.ds(cntb, _L)], bpk, mask=bmask)
            upc = plsc.all_reduce_population_count(umask)
            bpc = plsc.all_reduce_population_count(bmask)
            return (cntu + upc[0], cntb + bpc[0])

        return lax.fori_loop(0, _SCAN // _L, vec_body, cnt)

    return lax.fori_loop(0, _B // _SCAN, chunk_body, (0, 0))


def _window(list_v, sub_v, src_v, acc_sh, wvec, iota, n,
            rel_lo, rel_w, pan, col_off, col_hi):
    nvec = lax.div(n + (_L - 1), _L)
    dump = jnp.full((_L,), _B, jnp.int32)

    def blk_body(blk, carry):
        base = blk * _SUB

        def sub_body(k, cnt2):
            pk = list_v[pl.ds(base + k * _L, _L)]
            r2 = lax.shift_right_logical(pk, 15) - rel_lo
            mask = plsc.bitcast(r2, jnp.uint32) < plsc.bitcast(
                jnp.full((_L,), rel_w, jnp.int32), jnp.uint32)
            valid = (base + k * _L + iota) < n
            mask = jnp.logical_and(mask, valid)
            plsc.store_compressed(sub_v.at[pl.ds(cnt2, _L)], pk, mask=mask)
            pc = plsc.all_reduce_population_count(mask)
            return cnt2 + pc[0]

        kvecs = jnp.minimum(nvec - blk * (_SUB // _L), _SUB // _L)
        cnt2 = lax.fori_loop(0, kvecs, sub_body, 0)
        sub_v[pl.ds(cnt2, _L)] = dump

        def grp_body(g, carry):
            pk = sub_v[pl.ds(g * _L, _L)]
            pos = jnp.bitwise_and(pk, 32767)
            col = lax.shift_right_logical(pk, 15) - rel_lo + col_off
            col = jnp.minimum(jnp.maximum(col, 0), col_hi)
            acc = jnp.zeros((_L,), jnp.float32)
            for j in range(_D):
                pv = plsc.load_gather(pan, [jnp.full((_L,), j, jnp.int32), col])
                acc = acc + pv * wvec[j]
            src_v[...] = acc
            pltpu.sync_copy(src_v, acc_sh.at[pos], add=True)
            return carry

        lax.fori_loop(0, lax.div(cnt2 + (_L - 1), _L), grp_body, 0)
        return carry

    lax.fori_loop(0, lax.div(n + (_SUB - 1), _SUB), blk_body, 0)


def _fire_tiles(tab_hbm, panel_v, psem, lo, hi_main, p, slot):
    start = pl.multiple_of(jnp.minimum(lo + p * _PC, hi_main - _PC), 128)
    for jt in range(_D // 8):
        for ct in range(_PC // 128):
            pltpu.async_copy(
                tab_hbm.at[pl.ds(jt * 8, 8), pl.ds(start + ct * 128, 128)],
                panel_v.at[slot, pl.ds(jt * 8, 8), pl.ds(ct * 128, 128)],
                psem)


def _panel_pass(tab_hbm, lo, hi, tail_w, n, list_v, panel_v, sub_v, src_v,
                acc_sh, wvec, psem, iota, prefired=False):
    hi_main = jnp.bitwise_and(hi, ~127)
    n_panels = lax.div(hi_main - lo + (_PC - 1), _PC)

    def fire(p, slot):
        _fire_tiles(tab_hbm, panel_v, psem, lo, hi_main, p, slot)

    def drain(slot):
        for jt in range(_D // 8):
            for ct in range(_PC // 128):
                pltpu.make_async_copy(
                    tab_hbm.at[pl.ds(jt * 8, 8), pl.ds(0, 128)],
                    panel_v.at[slot, pl.ds(jt * 8, 8), pl.ds(ct * 128, 128)],
                    psem).wait()

    if not prefired:
        fire(0, 0)
    drain(0)

    def panel_body(p, carry):
        fire_pred = p + 1 < n_panels
        if prefired:
            fire_pred = jnp.logical_and(fire_pred, p >= 1)

        @pl.when(fire_pred)
        def _():
            fire(p + 1, (p + 1) % 2)

        start = jnp.minimum(lo + p * _PC, hi_main - _PC)
        rel_lo = p * _PC
        rel_w = jnp.minimum(rel_lo + _PC, hi_main - lo) - rel_lo
        _window(list_v, sub_v, src_v, acc_sh, wvec, iota, n,
                rel_lo, rel_w, panel_v.at[p % 2],
                (lo + rel_lo) - start, _PC - 1)
        @pl.when(p + 1 < n_panels)
        def _():
            drain((p + 1) % 2)
        return carry

    lax.fori_loop(0, n_panels, panel_body, 0)

    if tail_w:
        @pl.when(hi != hi_main)
        def _():
            tstart = pl.multiple_of(hi_main, 128)
            for jt in range(_D // 8):
                pltpu.sync_copy(
                    tab_hbm.at[pl.ds(jt * 8, 8), pl.ds(tstart, 128)],
                    panel_v.at[0, pl.ds(jt * 8, 8), pl.ds(0, 128)])
            _window(list_v, sub_v, src_v, acc_sh, wvec, iota, n,
                    hi_main - lo, hi - hi_main, panel_v.at[0],
                    0, 127)


def _phase1(users_hbm, books_hbm, ut_hbm, bt_hbm, wb_hbm,
            outp_hbm,
            stag_v, stagb_v, ulist_v, blist_v, panel_v, sub_v, src_v, w_v,
            zbuf_v, acc_sh, psem):
    wid = _wid()
    sid = lax.axis_index("s")
    core = lax.axis_index("c")
    ulo, uhi = _range_bounds(wid, _NU)
    blo, bhi = _range_bounds(wid, _NB)

    pltpu.sync_copy(wb_hbm, w_v)
    iota = lax.iota(jnp.int32, _L)

    uhi_main = jnp.bitwise_and(uhi, ~127)
    _fire_tiles(ut_hbm, panel_v, psem, ulo, uhi_main, 0, 0)
    _fire_tiles(ut_hbm, panel_v, psem, ulo, uhi_main, 1, 1)

    def z_body(k, carry):
        zbuf_v[pl.ds(k * _L, _L)] = jnp.zeros((_L,), jnp.float32)
        return carry
    lax.fori_loop(0, (_ACC // 16) // _L, z_body, 0)
    pltpu.sync_copy(zbuf_v, acc_sh.at[pl.ds(pl.multiple_of(sid * (_ACC // 16), 16), _ACC // 16)])
    plsc.subcore_barrier()

    nu, nb = _scan_compact(users_hbm, books_hbm, ulo, uhi - ulo,
                           blo, bhi - blo, stag_v, stagb_v,
                           ulist_v, blist_v, iota)

    wvecs = [w_v[pl.ds(k * _L, _L)] for k in range(4)]
    wu = [wvecs[j // _L][j % _L] for j in range(_D)]
    wk = [wvecs[2 + j // _L][j % _L] for j in range(_D)]

    _panel_pass(ut_hbm, ulo, uhi, _NU % 128, nu, ulist_v, panel_v, sub_v,
                src_v, acc_sh, wu, psem, iota, prefired=True)
    _panel_pass(bt_hbm, blo, bhi, _NB % 128, nb, blist_v, panel_v, sub_v,
                src_v, acc_sh, wk, psem, iota)

    plsc.subcore_barrier()
    off = pl.multiple_of(sid * (_ACC // 16), 16)
    pltpu.sync_copy(acc_sh.at[pl.ds(off, _ACC // 16)], zbuf_v)
    pltpu.sync_copy(
        zbuf_v,
        outp_hbm.at[pl.ds(pl.multiple_of(core * _ACC + off, 16), _ACC // 16)])


def _phase2(outp_hbm, wb_hbm, out_hbm, u_v, b_v, w_v):
    wid = _wid()
    base = wid * (_B // _NW)
    pltpu.sync_copy(outp_hbm.at[pl.ds(base, _B // _NW)], u_v)
    pltpu.sync_copy(outp_hbm.at[pl.ds(_ACC + base, _B // _NW)], b_v)
    pltpu.sync_copy(wb_hbm, w_v)
    bias = w_v[pl.ds(2 * _D, _L)][0]

    def body(k, carry):
        sl = pl.ds(k * _L, _L)
        u_v[sl] = u_v[sl] + b_v[sl] + bias
        return carry

    lax.fori_loop(0, (_B // _NW) // _L, body, 0)
    pltpu.sync_copy(u_v, out_hbm.at[pl.ds(base, _B // _NW)])


@jax.jit
def kernel(users, books, user_table, book_table, W, b):
    users = users.astype(jnp.int32)
    books = books.astype(jnp.int32)
    ut = user_table.T
    bt = book_table.T
    wb = jnp.concatenate(
        [W.reshape(-1), jnp.broadcast_to(b.reshape(-1)[0], (16,))]
    ).astype(jnp.float32)

    mesh = plsc.VectorSubcoreMesh(core_axis_name="c", subcore_axis_name="s")
    cap = _B + 2 * _L

    p1 = pl.kernel(
        _phase1,
        out_type=jax.ShapeDtypeStruct((2 * _ACC,), jnp.float32),
        mesh=mesh,
        compiler_params=pltpu.CompilerParams(needs_layout_passes=False),
        scratch_types=[
            pltpu.VMEM((_SCAN,), jnp.int32),
            pltpu.VMEM((_SCAN,), jnp.int32),
            pltpu.VMEM((cap,), jnp.int32),
            pltpu.VMEM((cap,), jnp.int32),
            pltpu.VMEM((2, _D, _PC), jnp.float32),
            pltpu.VMEM((_SUB + _L,), jnp.int32),
            pltpu.VMEM((_L,), jnp.float32),
            pltpu.VMEM((2 * _D + _L,), jnp.float32),
            pltpu.VMEM((_ACC // 16,), jnp.float32),
            pltpu.VMEM_SHARED((_ACC,), jnp.float32),
            pltpu.SemaphoreType.DMA,
        ],
    )
    outp = p1(users, books, ut, bt, wb)

    p2 = pl.kernel(
        _phase2,
        out_type=jax.ShapeDtypeStruct((_B,), jnp.float32),
        mesh=mesh,
        compiler_params=pltpu.CompilerParams(needs_layout_passes=False),
        scratch_types=[
            pltpu.VMEM((_B // _NW,), jnp.float32),
            pltpu.VMEM((_B // _NW,), jnp.float32),
            pltpu.VMEM((2 * _D + _L,), jnp.float32),
        ],
    )
    out = p2(outp, wb)
    return out.reshape(_B, 1)
